# baseline (speedup 1.0000x reference)
import jax
import jax.numpy as jnp
from jax import lax
from jax.experimental import pallas as pl
from jax.experimental.pallas import tpu as pltpu
from jax.experimental.pallas import tpu_sc as plsc

N_NODES = 10000
IN_DIM = 128
HID = 128
OUT_DIM = 64
N_EDGES = 320000
BN_INV = 1.0 / (1.0 + 1e-5) ** 0.5

NC = 2
NT = 16
CHUNK = 125
SUP = 40
CP_TILES = 10
CP_ROWS = N_NODES // CP_TILES
EPT = N_EDGES // (NC * NT)
NROW = EPT // CHUNK
R = 1000
GRID = N_NODES // R


def _mesh():
    return plsc.VectorSubcoreMesh(core_axis_name="c", subcore_axis_name="s")


HROWS = 80
HCAP = HROWS * 128
TRASH = HCAP - 1
DEG_ROWS = NC * NT * HROWS


def _deg_body(dst_hbm, out_hbm, hist1, di):
    cid = lax.axis_index("c")
    sid = lax.axis_index("s")
    wid = cid * NT + sid

    row0 = wid * HROWS
    pltpu.sync_copy(dst_hbm.at[pl.ds(row0, HROWS)], di)

    zeros = jnp.zeros((16,), jnp.float32)
    ones = jnp.full((16,), 1.0, jnp.float32)

    def zbody(j, carry):
        hist1[pl.ds(j * 16, 16)] = zeros
        return carry

    lax.fori_loop(0, HCAP // 16, zbody, 0)

    def body(j, carry):
        for k in range(8):
            idx = di[j, pl.ds(k * 16, 16)]
            plsc.addupdate_scatter(hist1, [idx], ones)
        return carry

    lax.fori_loop(0, HROWS, body, 0)
    pltpu.sync_copy(hist1, out_hbm.at[wid])


def _deg_call(dst128):
    k = pl.kernel(
        _deg_body,
        out_type=jax.ShapeDtypeStruct((NC * NT, HCAP), jnp.float32),
        mesh=_mesh(),
        scratch_types=[
            pltpu.VMEM((HCAP,), jnp.float32),
            pltpu.VMEM((HROWS, 128), jnp.int32),
        ],
        name="gcn_degree_hist",
        compiler_params=pltpu.CompilerParams(needs_layout_passes=False),
    )
    return k(dst128)


def _agg_body(y_hbm, src_hbm, dst_hbm, out_hbm,
              agg, si, di, buf0, buf1, sem0, sem1, ssem0, ssem1):
    cid = lax.axis_index("c")
    sid = lax.axis_index("s")
    r0 = sid * CP_ROWS

    @pl.when(sid < CP_TILES)
    def _zero():
        zeros = jnp.zeros((16,), jnp.float32)

        def zb(j, carry):
            for k in range(8):
                buf0[j, pl.ds(k * 16, 16)] = zeros
            return carry

        lax.fori_loop(0, CHUNK, zb, 0)
        for q in range(CP_ROWS // CHUNK):
            pltpu.sync_copy(buf0, agg.at[pl.ds(r0 + q * CHUNK, CHUNK)])

    row0 = cid * (N_EDGES // NC // CHUNK) + sid * NROW
    plsc.subcore_barrier()

    def outer(g, carry):
        pltpu.sync_copy(src_hbm.at[pl.ds(row0 + g * SUP, SUP)], si)
        pltpu.sync_copy(dst_hbm.at[pl.ds(row0 + g * SUP, SUP)], di)
        pltpu.async_copy(y_hbm.at[si.at[0]], buf0, sem0)

        def pair(p, c2):
            j0 = 2 * p
            j1 = j0 + 1

            @pl.when(p > 0)
            def _drain1():
                pltpu.make_async_copy(
                    buf1, agg.at[di.at[j1 - 2]], ssem1).wait()

            pltpu.async_copy(y_hbm.at[si.at[j1]], buf1, sem1)
            pltpu.make_async_copy(y_hbm.at[si.at[j0]], buf0, sem0).wait()
            pltpu.async_copy(buf0, agg.at[di.at[j0]], ssem0, add=True)
            pltpu.make_async_copy(y_hbm.at[si.at[j1]], buf1, sem1).wait()
            pltpu.async_copy(buf1, agg.at[di.at[j1]], ssem1, add=True)

            @pl.when(p < SUP // 2 - 1)
            def _next():
                pltpu.make_async_copy(
                    buf0, agg.at[di.at[j0]], ssem0).wait()
                pltpu.async_copy(y_hbm.at[si.at[j0 + 2]], buf0, sem0)

            return c2

        lax.fori_loop(0, SUP // 2, pair, 0)
        pltpu.make_async_copy(buf0, agg.at[di.at[SUP - 2]], ssem0).wait()
        pltpu.make_async_copy(buf1, agg.at[di.at[SUP - 1]], ssem1).wait()
        return carry

    lax.fori_loop(0, NROW // SUP, outer, 0)
    plsc.subcore_barrier()

    @pl.when(sid < CP_TILES)
    def _out():
        pltpu.sync_copy(agg.at[pl.ds(r0, CP_ROWS)],
                        out_hbm.at[pl.ds(cid * N_NODES + r0, CP_ROWS)])


def _agg_call(y, src2, dst2):
    k = pl.kernel(
        _agg_body,
        out_type=jax.ShapeDtypeStruct((NC * N_NODES, HID), jnp.float32),
        mesh=_mesh(),
        scratch_types=[
            pltpu.VMEM_SHARED((N_NODES, HID), jnp.float32),
            pltpu.VMEM((SUP, CHUNK), jnp.int32),
            pltpu.VMEM((SUP, CHUNK), jnp.int32),
            pltpu.VMEM((CHUNK, HID), jnp.float32),
            pltpu.VMEM((CHUNK, HID), jnp.float32),
            pltpu.SemaphoreType.DMA,
            pltpu.SemaphoreType.DMA,
            pltpu.SemaphoreType.DMA,
            pltpu.SemaphoreType.DMA,
        ],
        name="gcn_scatter_agg",
        compiler_params=pltpu.CompilerParams(needs_layout_passes=False),
    )
    return k(y, src2, dst2)


def _tcdis_body(degp_ref, out_ref):
    deg = jnp.sum(degp_ref[...], axis=0)[:N_NODES]
    out_ref[...] = lax.rsqrt(deg + 1.0)[:, None]


def _tc_dis(degp):
    return pl.pallas_call(
        _tcdis_body,
        grid=(1,),
        in_specs=[pl.BlockSpec((NC * NT, HCAP), lambda i: (0, 0))],
        out_specs=pl.BlockSpec((N_NODES, 1), lambda i: (0, 0)),
        out_shape=jax.ShapeDtypeStruct((N_NODES, 1), jnp.float32),
    )(degp)


def _tc1_body(dis_ref, x_ref, w_ref, y_ref):
    dis = dis_ref[...]
    xw = jnp.dot(x_ref[...], w_ref[...], preferred_element_type=jnp.float32)
    y_ref[...] = xw * dis


def _tc1(dis2, x, W1):
    return pl.pallas_call(
        _tc1_body,
        grid=(GRID,),
        in_specs=[
            pl.BlockSpec((R, 1), lambda i: (i, 0)),
            pl.BlockSpec((R, IN_DIM), lambda i: (i, 0)),
            pl.BlockSpec((IN_DIM, HID), lambda i: (0, 0)),
        ],
        out_specs=pl.BlockSpec((R, HID), lambda i: (i, 0)),
        out_shape=jax.ShapeDtypeStruct((N_NODES, HID), jnp.float32),
    )(dis2, x, W1)


def _tc2_body(dis_ref, y_ref, agg_ref, b_ref, g_ref, be_ref, w_ref, out_ref):
    dis = dis_ref[...]
    h = agg_ref[0] + agg_ref[1] + y_ref[...]
    h = dis * h + b_ref[...]
    h = h * (g_ref[...] * BN_INV) + be_ref[...]
    h = jnp.maximum(h, 0.0)
    out_ref[...] = jnp.dot(
        h, w_ref[...], preferred_element_type=jnp.float32) * dis


def _tc2(dis2, y1, agg3, b1, g1, be1, W2):
    return pl.pallas_call(
        _tc2_body,
        grid=(GRID,),
        in_specs=[
            pl.BlockSpec((R, 1), lambda i: (i, 0)),
            pl.BlockSpec((R, HID), lambda i: (i, 0)),
            pl.BlockSpec((2, R, HID), lambda i: (0, i, 0)),
            pl.BlockSpec((1, HID), lambda i: (0, 0)),
            pl.BlockSpec((1, HID), lambda i: (0, 0)),
            pl.BlockSpec((1, HID), lambda i: (0, 0)),
            pl.BlockSpec((HID, HID), lambda i: (0, 0)),
        ],
        out_specs=pl.BlockSpec((R, HID), lambda i: (i, 0)),
        out_shape=jax.ShapeDtypeStruct((N_NODES, HID), jnp.float32),
    )(dis2, y1, agg3, b1, g1, be1, W2)


def _tc3_body(dis_ref, y_ref, agg_ref, b_ref, g_ref, be_ref,
              cw1_ref, cb1_ref, cw2_ref, cb2_ref, out_ref):
    dis = dis_ref[...]
    h = agg_ref[0] + agg_ref[1] + y_ref[...]
    h = dis * h + b_ref[...]
    h = h * (g_ref[...] * BN_INV) + be_ref[...]
    h = jnp.maximum(h, 0.0)
    c = jnp.dot(h, cw1_ref[...], preferred_element_type=jnp.float32)
    c = jnp.maximum(c + cb1_ref[...], 0.0)
    out_ref[...] = jnp.dot(
        c, cw2_ref[...], preferred_element_type=jnp.float32) + cb2_ref[...]


def _tc3(dis2, y2, agg3, b2, g2, be2, cW1, cb1, cW2, cb2):
    return pl.pallas_call(
        _tc3_body,
        grid=(GRID,),
        in_specs=[
            pl.BlockSpec((R, 1), lambda i: (i, 0)),
            pl.BlockSpec((R, HID), lambda i: (i, 0)),
            pl.BlockSpec((2, R, HID), lambda i: (0, i, 0)),
            pl.BlockSpec((1, HID), lambda i: (0, 0)),
            pl.BlockSpec((1, HID), lambda i: (0, 0)),
            pl.BlockSpec((1, HID), lambda i: (0, 0)),
            pl.BlockSpec((HID, OUT_DIM), lambda i: (0, 0)),
            pl.BlockSpec((1, OUT_DIM), lambda i: (0, 0)),
            pl.BlockSpec((OUT_DIM, OUT_DIM), lambda i: (0, 0)),
            pl.BlockSpec((1, OUT_DIM), lambda i: (0, 0)),
        ],
        out_specs=pl.BlockSpec((R, OUT_DIM), lambda i: (i, 0)),
        out_shape=jax.ShapeDtypeStruct((N_NODES, OUT_DIM), jnp.float32),
    )(dis2, y2, agg3, b2, g2, be2, cW1, cb1, cW2, cb2)


def kernel(x, edge_index, W1, b1, W2, b2, bn1_g, bn1_b, bn2_g, bn2_b,
           cW1, cb1, cW2, cb2):
    src = edge_index[0].astype(jnp.int32)
    dst = edge_index[1].astype(jnp.int32)
    src2 = src.reshape(-1, CHUNK)
    dst2 = dst.reshape(-1, CHUNK)
    pad = jnp.full((DEG_ROWS * 128 - N_EDGES,), TRASH, jnp.int32)
    dst128 = jnp.concatenate([dst, pad]).reshape(DEG_ROWS, 128)
    degp = _deg_call(dst128)
    dis2 = _tc_dis(degp)

    y1 = _tc1(dis2, x, W1)
    agg1 = _agg_call(y1, src2, dst2).reshape(2, N_NODES, HID)
    y2 = _tc2(dis2, y1, agg1, b1.reshape(1, -1), bn1_g.reshape(1, -1),
              bn1_b.reshape(1, -1), W2)
    agg2 = _agg_call(y2, src2, dst2).reshape(2, N_NODES, HID)
    out = _tc3(dis2, y2, agg2, b2.reshape(1, -1), bn2_g.reshape(1, -1),
               bn2_b.reshape(1, -1), cW1, cb1.reshape(1, -1), cW2,
               cb2.reshape(1, -1))
    return out


# speedup vs baseline: 1.2891x; 1.2891x over previous
import jax
import jax.numpy as jnp
from jax import lax
from jax.experimental import pallas as pl
from jax.experimental.pallas import tpu as pltpu
from jax.experimental.pallas import tpu_sc as plsc

N_NODES = 10000
IN_DIM = 128
HID = 128
OUT_DIM = 64
N_EDGES = 320000
BN_INV = 1.0 / (1.0 + 1e-5) ** 0.5

NC = 2
NT = 16
CHUNK = 125
SUP = 40
CP_TILES = 10
CP_ROWS = N_NODES // CP_TILES
EPT = N_EDGES // (NC * NT)
NROW = EPT // CHUNK
R = 1000
GRID = N_NODES // R


def _mesh():
    return plsc.VectorSubcoreMesh(core_axis_name="c", subcore_axis_name="s")


HROWS = 80
HCAP = HROWS * 128
TRASH = HCAP - 1
DEG_ROWS = NC * NT * HROWS


def _deg_body(dst_hbm, out_hbm, hist1, di):
    cid = lax.axis_index("c")
    sid = lax.axis_index("s")
    wid = cid * NT + sid

    row0 = wid * HROWS
    pltpu.sync_copy(dst_hbm.at[pl.ds(row0, HROWS)], di)

    zeros = jnp.zeros((16,), jnp.float32)
    ones = jnp.full((16,), 1.0, jnp.float32)

    def zbody(j, carry):
        hist1[pl.ds(j * 16, 16)] = zeros
        return carry

    lax.fori_loop(0, HCAP // 16, zbody, 0)

    def body(j, carry):
        for k in range(8):
            idx = di[j, pl.ds(k * 16, 16)]
            plsc.addupdate_scatter(hist1, [idx], ones)
        return carry

    lax.fori_loop(0, HROWS, body, 0)
    pltpu.sync_copy(hist1, out_hbm.at[wid])


def _deg_call(dst128):
    k = pl.kernel(
        _deg_body,
        out_type=jax.ShapeDtypeStruct((NC * NT, HCAP), jnp.float32),
        mesh=_mesh(),
        scratch_types=[
            pltpu.VMEM((HCAP,), jnp.float32),
            pltpu.VMEM((HROWS, 128), jnp.int32),
        ],
        name="gcn_degree_hist",
        compiler_params=pltpu.CompilerParams(needs_layout_passes=False),
    )
    return k(dst128)


def _agg_body(y_hbm, src_hbm, dst_hbm, out_hbm,
              agg, si, di, buf0, buf1, sem0, sem1):
    cid = lax.axis_index("c")
    sid = lax.axis_index("s")
    r0 = sid * CP_ROWS

    @pl.when(sid < CP_TILES)
    def _zero():
        zeros = jnp.zeros((16,), jnp.float32)

        def zb(j, carry):
            for k in range(8):
                buf0[j, pl.ds(k * 16, 16)] = zeros
            return carry

        lax.fori_loop(0, CHUNK, zb, 0)
        for q in range(CP_ROWS // CHUNK):
            pltpu.sync_copy(buf0, agg.at[pl.ds(r0 + q * CHUNK, CHUNK)])

    row0 = cid * (N_EDGES // NC // CHUNK) + sid * NROW
    plsc.subcore_barrier()

    def outer(g, carry):
        pltpu.sync_copy(src_hbm.at[pl.ds(row0 + g * SUP, SUP)], si)
        pltpu.sync_copy(dst_hbm.at[pl.ds(row0 + g * SUP, SUP)], di)
        pltpu.async_copy(y_hbm.at[si.at[0]], buf0, sem0)

        def pair(p, c2):
            j0 = 2 * p
            pltpu.async_copy(y_hbm.at[si.at[j0 + 1]], buf1, sem1)
            pltpu.make_async_copy(y_hbm.at[si.at[j0]], buf0, sem0).wait()
            pltpu.sync_copy(buf0, agg.at[di.at[j0]], add=True)

            @pl.when(p < SUP // 2 - 1)
            def _next():
                pltpu.async_copy(y_hbm.at[si.at[j0 + 2]], buf0, sem0)

            pltpu.make_async_copy(y_hbm.at[si.at[j0 + 1]], buf1, sem1).wait()
            pltpu.sync_copy(buf1, agg.at[di.at[j0 + 1]], add=True)
            return c2

        lax.fori_loop(0, SUP // 2, pair, 0)
        return carry

    lax.fori_loop(0, NROW // SUP, outer, 0)
    plsc.subcore_barrier()

    @pl.when(sid < CP_TILES)
    def _out():
        pltpu.sync_copy(agg.at[pl.ds(r0, CP_ROWS)],
                        out_hbm.at[pl.ds(cid * N_NODES + r0, CP_ROWS)])


def _agg_call(y, src2, dst2):
    k = pl.kernel(
        _agg_body,
        out_type=jax.ShapeDtypeStruct((NC * N_NODES, HID), jnp.float32),
        mesh=_mesh(),
        scratch_types=[
            pltpu.VMEM_SHARED((N_NODES, HID), jnp.float32),
            pltpu.VMEM((SUP, CHUNK), jnp.int32),
            pltpu.VMEM((SUP, CHUNK), jnp.int32),
            pltpu.VMEM((CHUNK, HID), jnp.float32),
            pltpu.VMEM((CHUNK, HID), jnp.float32),
            pltpu.SemaphoreType.DMA,
            pltpu.SemaphoreType.DMA,
        ],
        name="gcn_scatter_agg",
        compiler_params=pltpu.CompilerParams(needs_layout_passes=False),
    )
    return k(y, src2, dst2)


def _tc1_body(degp_ref, x_ref, w_ref, dis_ref, y_ref):
    deg = jnp.sum(degp_ref[...], axis=0)[:N_NODES]
    dis = lax.rsqrt(deg + 1.0)[:, None]
    dis_ref[...] = dis
    xw = jnp.dot(x_ref[...], w_ref[...], preferred_element_type=jnp.float32)
    y_ref[...] = xw * dis


def _tc1(degp, x, W1):
    return pl.pallas_call(
        _tc1_body,
        out_shape=[
            jax.ShapeDtypeStruct((N_NODES, 1), jnp.float32),
            jax.ShapeDtypeStruct((N_NODES, HID), jnp.float32),
        ],
    )(degp, x, W1)


def _tc2_body(dis_ref, y_ref, agg_ref, b_ref, g_ref, be_ref, w_ref, out_ref):
    dis = dis_ref[...]
    h = agg_ref[0] + agg_ref[1] + y_ref[...]
    h = dis * h + b_ref[...]
    h = h * (g_ref[...] * BN_INV) + be_ref[...]
    h = jnp.maximum(h, 0.0)
    out_ref[...] = jnp.dot(
        h, w_ref[...], preferred_element_type=jnp.float32) * dis


def _tc2(dis2, y1, agg3, b1, g1, be1, W2):
    return pl.pallas_call(
        _tc2_body,
        out_shape=jax.ShapeDtypeStruct((N_NODES, HID), jnp.float32),
    )(dis2, y1, agg3, b1, g1, be1, W2)


def _tc3_body(dis_ref, y_ref, agg_ref, b_ref, g_ref, be_ref,
              cw1_ref, cb1_ref, cw2_ref, cb2_ref, out_ref):
    dis = dis_ref[...]
    h = agg_ref[0] + agg_ref[1] + y_ref[...]
    h = dis * h + b_ref[...]
    h = h * (g_ref[...] * BN_INV) + be_ref[...]
    h = jnp.maximum(h, 0.0)
    c = jnp.dot(h, cw1_ref[...], preferred_element_type=jnp.float32)
    c = jnp.maximum(c + cb1_ref[...], 0.0)
    out_ref[...] = jnp.dot(
        c, cw2_ref[...], preferred_element_type=jnp.float32) + cb2_ref[...]


def _tc3(dis2, y2, agg3, b2, g2, be2, cW1, cb1, cW2, cb2):
    return pl.pallas_call(
        _tc3_body,
        out_shape=jax.ShapeDtypeStruct((N_NODES, OUT_DIM), jnp.float32),
    )(dis2, y2, agg3, b2, g2, be2, cW1, cb1, cW2, cb2)


def kernel(x, edge_index, W1, b1, W2, b2, bn1_g, bn1_b, bn2_g, bn2_b,
           cW1, cb1, cW2, cb2):
    src = edge_index[0].astype(jnp.int32)
    dst = edge_index[1].astype(jnp.int32)
    src2 = src.reshape(-1, CHUNK)
    dst2 = dst.reshape(-1, CHUNK)
    pad = jnp.full((DEG_ROWS * 128 - N_EDGES,), TRASH, jnp.int32)
    dst128 = jnp.concatenate([dst, pad]).reshape(DEG_ROWS, 128)
    degp = _deg_call(dst128)
    dis2, y1 = _tc1(degp, x, W1)
    agg1 = _agg_call(y1, src2, dst2).reshape(2, N_NODES, HID)
    y2 = _tc2(dis2, y1, agg1, b1.reshape(1, -1), bn1_g.reshape(1, -1),
              bn1_b.reshape(1, -1), W2)
    agg2 = _agg_call(y2, src2, dst2).reshape(2, N_NODES, HID)
    out = _tc3(dis2, y2, agg2, b2.reshape(1, -1), bn2_g.reshape(1, -1),
               bn2_b.reshape(1, -1), cW1, cb1.reshape(1, -1), cW2,
               cb2.reshape(1, -1))
    return out
